# baseline (speedup 1.0000x reference)
import functools

import jax
import jax.numpy as jnp
from jax import lax
from jax.experimental import pallas as pl
from jax.experimental.pallas import tpu as pltpu
from jax.experimental.pallas import tpu_sc as plsc

_N = 16384
_D = 64
_K = 4096
_RT = 512
_W = 32


def _norms_body(x_ref, y_ref, ox_ref, oy_ref):
    for ref, out in ((x_ref, ox_ref), (y_ref, oy_ref)):
        t = ref[...].T
        sq = t * t
        P = sq[0:8, :]
        for g in range(1, 8):
            P = P + sq[8 * g:8 * g + 8, :]
        A1 = P[0:4, :] + P[4:8, :]
        A2 = A1[0:2, :] + A1[2:4, :]
        out[...] = A2[0, :] + A2[1, :]


def _norms(feat_select, feat_target):
    rt = 1024
    return pl.pallas_call(
        _norms_body,
        grid=(_N // rt,),
        in_specs=[pl.BlockSpec((rt, _D), lambda i: (i, 0)),
                  pl.BlockSpec((rt, _D), lambda i: (i, 0))],
        out_specs=[pl.BlockSpec((rt,), lambda i: (i,)),
                   pl.BlockSpec((rt,), lambda i: (i,))],
        out_shape=[jax.ShapeDtypeStruct((_N,), jnp.float32),
                   jax.ShapeDtypeStruct((_N,), jnp.float32)],
    )(feat_select, feat_target)


def _metric_body(x_ref, x2_ref, y_ref, y2_ref, o_ref):
    xt = x_ref[...]
    x2 = x2_ref[...][None, :]
    yt = y_ref[...]
    y2 = y2_ref[...][:, None]
    xy = jnp.dot(yt, xt.T, preferred_element_type=jnp.float32)
    d2 = (x2 + y2) - 2.0 * xy
    dist = jnp.sqrt(jnp.maximum(d2, 1e-12))
    D4 = dist.reshape(_W, (_N // _W) // 8, 8, _RT)
    P = jnp.sum(D4, axis=1)
    A1 = P[:, 0:4, :] + P[:, 4:8, :]
    A2 = A1[:, 0:2, :] + A1[:, 2:4, :]
    A3 = A2[:, 0, :] + A2[:, 1, :]
    m = A3[0]
    for w in range(1, _W):
        m = m + A3[w]
    o_ref[...] = m


def _metric(feat_select, feat_target, x2, y2):
    return pl.pallas_call(
        _metric_body,
        grid=(_N // _RT,),
        in_specs=[pl.BlockSpec((_RT, _D), lambda i: (i, 0)),
                  pl.BlockSpec((_RT,), lambda i: (i,)),
                  pl.BlockSpec((_N, _D), lambda i: (0, 0)),
                  pl.BlockSpec((_N,), lambda i: (0,))],
        out_specs=pl.BlockSpec((_RT,), lambda i: (i,)),
        out_shape=jax.ShapeDtypeStruct((_N,), jnp.float32),
    )(feat_select, x2, feat_target, y2)


def _select_body(k_ref, p_ref):
    keys = k_ref[...].reshape(128, 128)

    def bit_step(t, T):
        cand = T | (1 << (30 - t))
        n_ge = jnp.sum((keys >= cand).astype(jnp.int32))
        return jnp.where(n_ge >= _K, cand, T)

    T = jax.lax.fori_loop(0, 31, bit_step, jnp.int32(0))
    n_gt = jnp.sum((keys > T).astype(jnp.int32))
    ties_needed = _K - n_gt

    rr = lax.broadcasted_iota(jnp.int32, (128, 128), 0)
    cc = lax.broadcasted_iota(jnp.int32, (128, 128), 1)
    U = (rr < cc).astype(jnp.float32)

    def excl_prefix(mask_f32):
        pre = jnp.dot(mask_f32, U, preferred_element_type=jnp.float32)
        rows = jnp.sum(mask_f32, axis=-1)
        row_off = jnp.dot(rows[None, :], U, preferred_element_type=jnp.float32)[0]
        return row_off[:, None] + pre

    eq = keys == T
    eq_before = excl_prefix(eq.astype(jnp.float32)).astype(jnp.int32)
    sel = (keys > T) | (eq & (eq_before < ties_needed))
    pos = excl_prefix(sel.astype(jnp.float32)).astype(jnp.int32)
    p_ref[...] = jnp.where(sel, pos, _K).reshape(_N)


def _select(keys):
    return pl.pallas_call(
        _select_body,
        out_shape=jax.ShapeDtypeStruct((_N,), jnp.int32),
    )(keys)


_RANK_T = 1024


def _rank_body(ka_ref, kb_ref, r_ref):
    i = pl.program_id(0)
    ka = ka_ref[...][:, None]
    kb = kb_ref[...][None, :]
    a_idx = i * _RANK_T + lax.broadcasted_iota(jnp.int32, (_RANK_T, _K), 0)
    b_idx = lax.broadcasted_iota(jnp.int32, (_RANK_T, _K), 1)
    gt = (kb > ka) | ((kb == ka) & (b_idx < a_idx))
    r_ref[...] = jnp.sum(gt.astype(jnp.int32), axis=-1)


def _rank(cand_key):
    return pl.pallas_call(
        _rank_body,
        grid=(_K // _RANK_T,),
        in_specs=[pl.BlockSpec((_RANK_T,), lambda i: (i,)),
                  pl.BlockSpec((_K,), lambda i: (0,))],
        out_specs=pl.BlockSpec((_RANK_T,), lambda i: (i,)),
        out_shape=jax.ShapeDtypeStruct((_K,), jnp.int32),
    )(cand_key, cand_key)


_SC_INFO = plsc.get_sparse_core_info()
_NC = _SC_INFO.num_cores
_NS = _SC_INFO.num_subcores
_NW = _NC * _NS
_SC_MESH = plsc.VectorSubcoreMesh(core_axis_name="c", subcore_axis_name="s")
_CCH = _N // _NS
_FCH = _K // _NW
_ICH = _K // _NS


@functools.partial(
    pl.kernel, mesh=_SC_MESH,
    out_type=[jax.ShapeDtypeStruct((_K + 1,), jnp.int32),
              jax.ShapeDtypeStruct((_K + 1,), jnp.int32)],
    scratch_types=[pltpu.VMEM((_CCH,), jnp.int32),
                   pltpu.VMEM((_CCH,), jnp.int32),
                   pltpu.VMEM((_CCH,), jnp.int32),
                   pltpu.VMEM_SHARED((_K + 1,), jnp.int32),
                   pltpu.VMEM_SHARED((_K + 1,), jnp.int32)],
)
def _sc_compact(keys_hbm, p_hbm, iota_hbm, cidx_hbm, ckey_hbm,
                keys_v, p_v, iota_v, cidx_sh, ckey_sh):
    cid = lax.axis_index("c")
    sid = lax.axis_index("s")

    @pl.when(cid == 0)
    def _():
        base = sid * _CCH
        pltpu.sync_copy(p_hbm.at[pl.ds(base, _CCH)], p_v)
        pltpu.sync_copy(keys_hbm.at[pl.ds(base, _CCH)], keys_v)
        pltpu.sync_copy(iota_hbm.at[pl.ds(base, _CCH)], iota_v)
        pltpu.sync_copy(iota_v, cidx_sh.at[p_v])
        pltpu.sync_copy(keys_v, ckey_sh.at[p_v])
        plsc.subcore_barrier()
        ob = sid * (_K // _NS)
        pltpu.sync_copy(cidx_sh.at[pl.ds(ob, _K // _NS)],
                        cidx_hbm.at[pl.ds(ob, _K // _NS)])
        pltpu.sync_copy(ckey_sh.at[pl.ds(ob, _K // _NS)],
                        ckey_hbm.at[pl.ds(ob, _K // _NS)])


@functools.partial(
    pl.kernel, mesh=_SC_MESH,
    out_type=[jax.ShapeDtypeStruct((_K,), jnp.int32),
              jax.ShapeDtypeStruct((_K, 128), jnp.float32)],
    scratch_types=[pltpu.VMEM((_FCH,), jnp.int32),
                   pltpu.VMEM((_FCH,), jnp.int32),
                   pltpu.VMEM((_FCH, 128), jnp.float32),
                   pltpu.VMEM((_ICH,), jnp.int32),
                   pltpu.VMEM((_ICH,), jnp.int32),
                   pltpu.VMEM_SHARED((_K,), jnp.int32),
                   pltpu.SemaphoreType.DMA,
                   pltpu.SemaphoreType.DMA],
)
def _sc_final(feat_hbm, cidx_hbm, r_hbm, oidx_hbm, osel_hbm,
              cidx_v, r_v, rows_v, cidx2_v, r2_v, oidx_sh, sem1, sem2):
    cid = lax.axis_index("c")
    sid = lax.axis_index("s")
    wid = sid * _NC + cid
    base = wid * _FCH
    pltpu.sync_copy(cidx_hbm.at[pl.ds(base, _FCH)], cidx_v)
    pltpu.sync_copy(r_hbm.at[pl.ds(base, _FCH)], r_v)
    pltpu.async_copy(feat_hbm.at[cidx_v], rows_v, sem1).wait()
    pltpu.async_copy(rows_v, osel_hbm.at[r_v], sem2).wait()

    @pl.when(cid == 0)
    def _():
        base2 = sid * _ICH
        pltpu.sync_copy(cidx_hbm.at[pl.ds(base2, _ICH)], cidx2_v)
        pltpu.sync_copy(r_hbm.at[pl.ds(base2, _ICH)], r2_v)
        pltpu.sync_copy(cidx2_v, oidx_sh.at[r2_v])
        plsc.subcore_barrier()
        ob = sid * _ICH
        pltpu.sync_copy(oidx_sh.at[pl.ds(ob, _ICH)],
                        oidx_hbm.at[pl.ds(ob, _ICH)])


def kernel(feat_select, feat_target, k):
    x2, y2 = _norms(feat_select, feat_target)
    m = _metric(feat_select, feat_target, x2, y2)
    keys = jax.lax.bitcast_convert_type(m, jnp.int32)
    p = _select(keys)
    iota = jax.lax.iota(jnp.int32, _N)
    cidx_full, ckey_full = _sc_compact(keys, p, iota)
    cidx = cidx_full[:_K]
    ckey = ckey_full[:_K]
    r = _rank(ckey)
    feat128 = jnp.pad(feat_select, ((0, 0), (0, 128 - _D)))
    idx, sel128 = _sc_final(feat128, cidx, r)
    return sel128[:, :_D], idx


# speedup vs baseline: 1.0035x; 1.0035x over previous
import functools

import jax
import jax.numpy as jnp
from jax import lax
from jax.experimental import pallas as pl
from jax.experimental.pallas import tpu as pltpu
from jax.experimental.pallas import tpu_sc as plsc

_N = 16384
_D = 64
_K = 4096
_RT = 512
_W = 32


def _norms_body(x_ref, y_ref, ox_ref, oy_ref):
    for ref, out in ((x_ref, ox_ref), (y_ref, oy_ref)):
        t = ref[...].T
        sq = t * t
        P = sq[0:8, :]
        for g in range(1, 8):
            P = P + sq[8 * g:8 * g + 8, :]
        A1 = P[0:4, :] + P[4:8, :]
        A2 = A1[0:2, :] + A1[2:4, :]
        out[...] = A2[0, :] + A2[1, :]


def _norms(feat_select, feat_target):
    rt = 1024
    return pl.pallas_call(
        _norms_body,
        grid=(_N // rt,),
        in_specs=[pl.BlockSpec((rt, _D), lambda i: (i, 0)),
                  pl.BlockSpec((rt, _D), lambda i: (i, 0))],
        out_specs=[pl.BlockSpec((rt,), lambda i: (i,)),
                   pl.BlockSpec((rt,), lambda i: (i,))],
        out_shape=[jax.ShapeDtypeStruct((_N,), jnp.float32),
                   jax.ShapeDtypeStruct((_N,), jnp.float32)],
    )(feat_select, feat_target)


def _metric_body(x_ref, x2_ref, y_ref, y2_ref, o_ref):
    xt = x_ref[...]
    x2 = x2_ref[...][None, :]
    yt = y_ref[...]
    y2 = y2_ref[...][:, None]
    xy = jnp.dot(yt, xt.T, preferred_element_type=jnp.float32)
    d2 = (x2 + y2) - 2.0 * xy
    dist = jnp.sqrt(jnp.maximum(d2, 1e-12))
    D4 = dist.reshape(_W, (_N // _W) // 8, 8, _RT)
    P = jnp.sum(D4, axis=1)
    A1 = P[:, 0:4, :] + P[:, 4:8, :]
    A2 = A1[:, 0:2, :] + A1[:, 2:4, :]
    A3 = A2[:, 0, :] + A2[:, 1, :]
    m = A3[0]
    for w in range(1, _W):
        m = m + A3[w]
    o_ref[...] = m


def _metric(feat_select, feat_target, x2, y2):
    return pl.pallas_call(
        _metric_body,
        grid=(_N // _RT,),
        in_specs=[pl.BlockSpec((_RT, _D), lambda i: (i, 0)),
                  pl.BlockSpec((_RT,), lambda i: (i,)),
                  pl.BlockSpec((_N, _D), lambda i: (0, 0)),
                  pl.BlockSpec((_N,), lambda i: (0,))],
        out_specs=pl.BlockSpec((_RT,), lambda i: (i,)),
        out_shape=jax.ShapeDtypeStruct((_N,), jnp.float32),
    )(feat_select, x2, feat_target, y2)


def _select_body(k_ref, p_ref):
    keys = k_ref[...].reshape(128, 128)

    def bit_step(t, T):
        cand = T | (1 << (30 - t))
        n_ge = jnp.sum((keys >= cand).astype(jnp.int32))
        return jnp.where(n_ge >= _K, cand, T)

    T = jax.lax.fori_loop(0, 31, bit_step, jnp.int32(0))
    n_gt = jnp.sum((keys > T).astype(jnp.int32))
    ties_needed = _K - n_gt

    rr = lax.broadcasted_iota(jnp.int32, (128, 128), 0)
    cc = lax.broadcasted_iota(jnp.int32, (128, 128), 1)
    U = (rr < cc).astype(jnp.float32)

    def excl_prefix(mask_f32):
        pre = jnp.dot(mask_f32, U, preferred_element_type=jnp.float32)
        rows = jnp.sum(mask_f32, axis=-1)
        row_off = jnp.dot(rows[None, :], U, preferred_element_type=jnp.float32)[0]
        return row_off[:, None] + pre

    eq = keys == T
    eq_before = excl_prefix(eq.astype(jnp.float32)).astype(jnp.int32)
    sel = (keys > T) | (eq & (eq_before < ties_needed))
    pos = excl_prefix(sel.astype(jnp.float32)).astype(jnp.int32)
    p_ref[...] = jnp.where(sel, pos, _K).reshape(_N)


def _select(keys):
    return pl.pallas_call(
        _select_body,
        out_shape=jax.ShapeDtypeStruct((_N,), jnp.int32),
    )(keys)


_RANK_T = 1024


def _rank_body(ka_ref, kb_ref, r_ref):
    i = pl.program_id(0)
    ka = ka_ref[...][:, None]
    kb = kb_ref[...][None, :]
    a_idx = i * _RANK_T + lax.broadcasted_iota(jnp.int32, (_RANK_T, _K), 0)
    b_idx = lax.broadcasted_iota(jnp.int32, (_RANK_T, _K), 1)
    gt = (kb > ka) | ((kb == ka) & (b_idx < a_idx))
    r_ref[...] = jnp.sum(gt.astype(jnp.int32), axis=-1)


def _rank(cand_key):
    return pl.pallas_call(
        _rank_body,
        grid=(_K // _RANK_T,),
        in_specs=[pl.BlockSpec((_RANK_T,), lambda i: (i,)),
                  pl.BlockSpec((_K,), lambda i: (0,))],
        out_specs=pl.BlockSpec((_RANK_T,), lambda i: (i,)),
        out_shape=jax.ShapeDtypeStruct((_K,), jnp.int32),
    )(cand_key, cand_key)


_SC_INFO = plsc.get_sparse_core_info()
_NC = _SC_INFO.num_cores
_NS = _SC_INFO.num_subcores
_NW = _NC * _NS
_SC_MESH = plsc.VectorSubcoreMesh(core_axis_name="c", subcore_axis_name="s")
_CCH = _N // _NS
_FCH = _K // _NW
_ICH = _K // _NS


@functools.partial(
    pl.kernel, mesh=_SC_MESH,
    out_type=[jax.ShapeDtypeStruct((_K + 1,), jnp.int32),
              jax.ShapeDtypeStruct((_K + 1,), jnp.int32)],
    scratch_types=[pltpu.VMEM((_CCH // 128, 128), jnp.int32),
                   pltpu.VMEM((_CCH // 128, 128), jnp.int32),
                   pltpu.VMEM((_CCH // 128, 128), jnp.int32),
                   pltpu.VMEM_SHARED((_K + 1,), jnp.int32),
                   pltpu.VMEM_SHARED((_K + 1,), jnp.int32)],
)
def _sc_compact(keys_hbm, p_hbm, iota_hbm, cidx_hbm, ckey_hbm,
                keys_v, p_v, iota_v, cidx_sh, ckey_sh):
    cid = lax.axis_index("c")
    sid = lax.axis_index("s")
    nrows = _CCH // 128

    @pl.when(cid == 0)
    def _():
        base = sid * nrows
        pltpu.sync_copy(p_hbm.at[pl.ds(base, nrows)], p_v)
        pltpu.sync_copy(keys_hbm.at[pl.ds(base, nrows)], keys_v)
        pltpu.sync_copy(iota_hbm.at[pl.ds(base, nrows)], iota_v)
        for j in range(nrows):
            pltpu.sync_copy(iota_v.at[j], cidx_sh.at[p_v.at[j]])
            pltpu.sync_copy(keys_v.at[j], ckey_sh.at[p_v.at[j]])
        plsc.subcore_barrier()
        ob = sid * (_K // _NS)
        pltpu.sync_copy(cidx_sh.at[pl.ds(ob, _K // _NS)],
                        cidx_hbm.at[pl.ds(ob, _K // _NS)])
        pltpu.sync_copy(ckey_sh.at[pl.ds(ob, _K // _NS)],
                        ckey_hbm.at[pl.ds(ob, _K // _NS)])


@functools.partial(
    pl.kernel, mesh=_SC_MESH,
    out_type=[jax.ShapeDtypeStruct((_K,), jnp.int32),
              jax.ShapeDtypeStruct((_K, 128), jnp.float32)],
    scratch_types=[pltpu.VMEM((_FCH,), jnp.int32),
                   pltpu.VMEM((_FCH,), jnp.int32),
                   pltpu.VMEM((_FCH, 128), jnp.float32),
                   pltpu.VMEM((_ICH // 128, 128), jnp.int32),
                   pltpu.VMEM((_ICH // 128, 128), jnp.int32),
                   pltpu.VMEM_SHARED((_K,), jnp.int32),
                   pltpu.SemaphoreType.DMA,
                   pltpu.SemaphoreType.DMA],
)
def _sc_final(feat_hbm, cidx_hbm, r_hbm, oidx_hbm, osel_hbm,
              cidx_v, r_v, rows_v, cidx2_v, r2_v, oidx_sh, sem1, sem2):
    cid = lax.axis_index("c")
    sid = lax.axis_index("s")
    wid = sid * _NC + cid
    pltpu.sync_copy(cidx_hbm.at[wid], cidx_v)
    pltpu.sync_copy(r_hbm.at[wid], r_v)
    pltpu.async_copy(feat_hbm.at[cidx_v], rows_v, sem1).wait()
    pltpu.async_copy(rows_v, osel_hbm.at[r_v], sem2).wait()

    @pl.when(cid == 0)
    def _():
        nrows = _ICH // 128
        base2 = sid * nrows
        pltpu.sync_copy(cidx_hbm.at[pl.ds(base2, nrows)], cidx2_v)
        pltpu.sync_copy(r_hbm.at[pl.ds(base2, nrows)], r2_v)
        for j in range(nrows):
            pltpu.sync_copy(cidx2_v.at[j], oidx_sh.at[r2_v.at[j]])
        plsc.subcore_barrier()
        ob = sid * _ICH
        pltpu.sync_copy(oidx_sh.at[pl.ds(ob, _ICH)],
                        oidx_hbm.at[pl.ds(ob, _ICH)])


def kernel(feat_select, feat_target, k):
    x2, y2 = _norms(feat_select, feat_target)
    m = _metric(feat_select, feat_target, x2, y2)
    keys = jax.lax.bitcast_convert_type(m, jnp.int32)
    p = _select(keys)
    iota = jax.lax.iota(jnp.int32, _N)
    cidx_full, ckey_full = _sc_compact(keys.reshape(_N // 128, 128),
                                       p.reshape(_N // 128, 128),
                                       iota.reshape(_N // 128, 128))
    cidx = cidx_full[:_K]
    ckey = ckey_full[:_K]
    r = _rank(ckey)
    feat128 = jnp.pad(feat_select, ((0, 0), (0, 128 - _D)))
    idx, sel128 = _sc_final(feat128, cidx.reshape(_K // 128, 128),
                            r.reshape(_K // 128, 128))
    return sel128[:, :_D], idx


# speedup vs baseline: 1.0065x; 1.0030x over previous
import functools

import jax
import jax.numpy as jnp
from jax import lax
from jax.experimental import pallas as pl
from jax.experimental.pallas import tpu as pltpu
from jax.experimental.pallas import tpu_sc as plsc

_N = 16384
_D = 64
_K = 4096
_RT = 512
_W = 32


def _norms_body(x_ref, y_ref, ox_ref, oy_ref):
    for ref, out in ((x_ref, ox_ref), (y_ref, oy_ref)):
        t = ref[...].T
        sq = t * t
        P = sq[0:8, :]
        for g in range(1, 8):
            P = P + sq[8 * g:8 * g + 8, :]
        A1 = P[0:4, :] + P[4:8, :]
        A2 = A1[0:2, :] + A1[2:4, :]
        out[...] = A2[0, :] + A2[1, :]


def _norms(feat_select, feat_target):
    rt = 1024
    return pl.pallas_call(
        _norms_body,
        grid=(_N // rt,),
        in_specs=[pl.BlockSpec((rt, _D), lambda i: (i, 0)),
                  pl.BlockSpec((rt, _D), lambda i: (i, 0))],
        out_specs=[pl.BlockSpec((rt,), lambda i: (i,)),
                   pl.BlockSpec((rt,), lambda i: (i,))],
        out_shape=[jax.ShapeDtypeStruct((_N,), jnp.float32),
                   jax.ShapeDtypeStruct((_N,), jnp.float32)],
    )(feat_select, feat_target)


def _metric_body(x_ref, x2_ref, y_ref, y2_ref, o_ref):
    xt = x_ref[...]
    x2 = x2_ref[...][None, :]
    yt = y_ref[...]
    y2 = y2_ref[...][:, None]
    xy = jnp.dot(yt, xt.T, preferred_element_type=jnp.float32)
    d2 = (x2 + y2) - 2.0 * xy
    dist = jnp.sqrt(jnp.maximum(d2, 1e-12))
    D4 = dist.reshape(_W, (_N // _W) // 8, 8, _RT)
    P = jnp.sum(D4, axis=1)
    A1 = P[:, 0:4, :] + P[:, 4:8, :]
    A2 = A1[:, 0:2, :] + A1[:, 2:4, :]
    A3 = A2[:, 0, :] + A2[:, 1, :]
    m = A3[0]
    for w in range(1, _W):
        m = m + A3[w]
    o_ref[...] = m


def _metric(feat_select, feat_target, x2, y2):
    return pl.pallas_call(
        _metric_body,
        grid=(_N // _RT,),
        in_specs=[pl.BlockSpec((_RT, _D), lambda i: (i, 0)),
                  pl.BlockSpec((_RT,), lambda i: (i,)),
                  pl.BlockSpec((_N, _D), lambda i: (0, 0)),
                  pl.BlockSpec((_N,), lambda i: (0,))],
        out_specs=pl.BlockSpec((_RT,), lambda i: (i,)),
        out_shape=jax.ShapeDtypeStruct((_N,), jnp.float32),
    )(feat_select, x2, feat_target, y2)


def _select_body(k_ref, p_ref):
    keys = k_ref[...].reshape(128, 128)

    def bit_step(t, T):
        cand = T | (1 << (30 - t))
        n_ge = jnp.sum((keys >= cand).astype(jnp.int32))
        return jnp.where(n_ge >= _K, cand, T)

    T = jax.lax.fori_loop(0, 31, bit_step, jnp.int32(0))
    n_gt = jnp.sum((keys > T).astype(jnp.int32))
    ties_needed = _K - n_gt

    rr = lax.broadcasted_iota(jnp.int32, (128, 128), 0)
    cc = lax.broadcasted_iota(jnp.int32, (128, 128), 1)
    U = (rr < cc).astype(jnp.float32)

    def excl_prefix(mask_f32):
        pre = jnp.dot(mask_f32, U, preferred_element_type=jnp.float32)
        rows = jnp.sum(mask_f32, axis=-1)
        row_off = jnp.dot(rows[None, :], U, preferred_element_type=jnp.float32)[0]
        return row_off[:, None] + pre

    eq = keys == T
    eq_before = excl_prefix(eq.astype(jnp.float32)).astype(jnp.int32)
    sel = (keys > T) | (eq & (eq_before < ties_needed))
    pos = excl_prefix(sel.astype(jnp.float32)).astype(jnp.int32)
    p_ref[...] = jnp.where(sel, pos, _K).reshape(_N)


def _select(keys):
    return pl.pallas_call(
        _select_body,
        out_shape=jax.ShapeDtypeStruct((_N,), jnp.int32),
    )(keys)


_RANK_T = 1024


def _rank_body(ka_ref, kb_ref, r_ref):
    i = pl.program_id(0)
    ka = ka_ref[...][:, None]
    kb = kb_ref[...][None, :]
    a_idx = i * _RANK_T + lax.broadcasted_iota(jnp.int32, (_RANK_T, _K), 0)
    b_idx = lax.broadcasted_iota(jnp.int32, (_RANK_T, _K), 1)
    gt = (kb > ka) | ((kb == ka) & (b_idx < a_idx))
    r_ref[...] = jnp.sum(gt.astype(jnp.int32), axis=-1)


def _rank(cand_key):
    return pl.pallas_call(
        _rank_body,
        grid=(_K // _RANK_T,),
        in_specs=[pl.BlockSpec((_RANK_T,), lambda i: (i,)),
                  pl.BlockSpec((_K,), lambda i: (0,))],
        out_specs=pl.BlockSpec((_RANK_T,), lambda i: (i,)),
        out_shape=jax.ShapeDtypeStruct((_K,), jnp.int32),
    )(cand_key, cand_key)


_SC_INFO = plsc.get_sparse_core_info()
_NC = _SC_INFO.num_cores
_NS = _SC_INFO.num_subcores
_NW = _NC * _NS
_SC_MESH = plsc.VectorSubcoreMesh(core_axis_name="c", subcore_axis_name="s")
_CCH = _N // _NS
_FCH = _K // _NW
_ICH = _K // _NS


@functools.partial(
    pl.kernel, mesh=_SC_MESH,
    out_type=[jax.ShapeDtypeStruct((_K + 1,), jnp.int32),
              jax.ShapeDtypeStruct((_K + 1,), jnp.int32)],
    scratch_types=[pltpu.VMEM((_CCH // 128, 128), jnp.int32),
                   pltpu.VMEM((_CCH // 128, 128), jnp.int32),
                   pltpu.VMEM((_CCH // 128, 128), jnp.int32),
                   pltpu.VMEM_SHARED((_K + 1,), jnp.int32),
                   pltpu.VMEM_SHARED((_K + 1,), jnp.int32)],
)
def _sc_compact(keys_hbm, p_hbm, iota_hbm, cidx_hbm, ckey_hbm,
                keys_v, p_v, iota_v, cidx_sh, ckey_sh):
    cid = lax.axis_index("c")
    sid = lax.axis_index("s")
    nrows = _CCH // 128

    @pl.when(cid == 0)
    def _():
        base = sid * nrows
        pltpu.sync_copy(p_hbm.at[pl.ds(base, nrows)], p_v)
        pltpu.sync_copy(keys_hbm.at[pl.ds(base, nrows)], keys_v)
        pltpu.sync_copy(iota_hbm.at[pl.ds(base, nrows)], iota_v)
        for j in range(nrows):
            pltpu.sync_copy(iota_v.at[j], cidx_sh.at[p_v.at[j]])
            pltpu.sync_copy(keys_v.at[j], ckey_sh.at[p_v.at[j]])
        plsc.subcore_barrier()
        ob = sid * (_K // _NS)
        pltpu.sync_copy(cidx_sh.at[pl.ds(ob, _K // _NS)],
                        cidx_hbm.at[pl.ds(ob, _K // _NS)])
        pltpu.sync_copy(ckey_sh.at[pl.ds(ob, _K // _NS)],
                        ckey_hbm.at[pl.ds(ob, _K // _NS)])


@functools.partial(
    pl.kernel, mesh=_SC_MESH,
    out_type=jax.ShapeDtypeStruct((_K, 128), jnp.float32),
    scratch_types=[pltpu.VMEM((_FCH,), jnp.int32),
                   pltpu.VMEM((_FCH, 128), jnp.float32),
                   pltpu.SemaphoreType.DMA],
)
def _sc_gather(feat_hbm, cidx_hbm, rowsg_hbm, cidx_v, rows_v, sem1):
    cid = lax.axis_index("c")
    sid = lax.axis_index("s")
    wid = sid * _NC + cid
    pltpu.sync_copy(cidx_hbm.at[wid], cidx_v)
    pltpu.async_copy(feat_hbm.at[cidx_v], rows_v, sem1).wait()
    pltpu.sync_copy(rows_v, rowsg_hbm.at[pl.ds(wid * _FCH, _FCH)])


@functools.partial(
    pl.kernel, mesh=_SC_MESH,
    out_type=[jax.ShapeDtypeStruct((_K,), jnp.int32),
              jax.ShapeDtypeStruct((_K, 128), jnp.float32)],
    scratch_types=[pltpu.VMEM((_FCH,), jnp.int32),
                   pltpu.VMEM((_FCH, 128), jnp.float32),
                   pltpu.VMEM((_ICH // 128, 128), jnp.int32),
                   pltpu.VMEM((_ICH // 128, 128), jnp.int32),
                   pltpu.VMEM_SHARED((_K,), jnp.int32),
                   pltpu.SemaphoreType.DMA],
)
def _sc_scatter(rowsg_hbm, cidx_hbm, r_hbm, oidx_hbm, osel_hbm,
                r_v, rows_v, cidx2_v, r2_v, oidx_sh, sem1):
    cid = lax.axis_index("c")
    sid = lax.axis_index("s")
    wid = sid * _NC + cid
    pltpu.sync_copy(r_hbm.at[wid], r_v)
    pltpu.sync_copy(rowsg_hbm.at[pl.ds(wid * _FCH, _FCH)], rows_v)
    pltpu.async_copy(rows_v, osel_hbm.at[r_v], sem1).wait()

    @pl.when(cid == 0)
    def _():
        nrows = _ICH // 128
        base2 = sid * nrows
        pltpu.sync_copy(cidx_hbm.at[pl.ds(base2, nrows)], cidx2_v)
        pltpu.sync_copy(r_hbm.at[pl.ds(base2, nrows)], r2_v)
        for j in range(nrows):
            pltpu.sync_copy(cidx2_v.at[j], oidx_sh.at[r2_v.at[j]])
        plsc.subcore_barrier()
        ob = sid * _ICH
        pltpu.sync_copy(oidx_sh.at[pl.ds(ob, _ICH)],
                        oidx_hbm.at[pl.ds(ob, _ICH)])


def kernel(feat_select, feat_target, k):
    x2, y2 = _norms(feat_select, feat_target)
    m = _metric(feat_select, feat_target, x2, y2)
    keys = jax.lax.bitcast_convert_type(m, jnp.int32)
    p = _select(keys)
    iota = jax.lax.iota(jnp.int32, _N)
    cidx_full, ckey_full = _sc_compact(keys.reshape(_N // 128, 128),
                                       p.reshape(_N // 128, 128),
                                       iota.reshape(_N // 128, 128))
    cidx = cidx_full[:_K]
    ckey = ckey_full[:_K]
    feat128 = jnp.pad(feat_select, ((0, 0), (0, 128 - _D)))
    cidx2d = cidx.reshape(_K // 128, 128)
    rows_g = _sc_gather(feat128, cidx2d)
    r = _rank(ckey)
    idx, sel128 = _sc_scatter(rows_g, cidx2d, r.reshape(_K // 128, 128))
    return sel128[:, :_D], idx
